# baseline (speedup 1.0000x reference)
import functools

import jax
import jax.numpy as jnp
from jax import lax
from jax.experimental import pallas as pl
from jax.experimental.pallas import tpu as pltpu
from jax.experimental.pallas import tpu_sc as plsc

N = 10000
E = 320000
D = 128
R = 3

NP = 10240
NW = 32
NS = 16
EW = 10240
EP = NW * EW
K = 128
NCHUNK = EW // K
ROWS_PER_TILE = NP // NS
H4 = 4 * NP
F32 = jnp.float32
I32 = jnp.int32

_mesh = lambda: plsc.VectorSubcoreMesh(core_axis_name="c", subcore_axis_name="s")
_SC_PARAMS = pltpu.CompilerParams(needs_layout_passes=False)


def _wid():
    return lax.axis_index("s") * 2 + lax.axis_index("c")


def _k0_body(src_h, dst_h, et_h, ho_out, hi_out, g4_out, d4_out, gm_out,
             s_v, d_v, t_v, ho_v, hi_v):
    wid = _wid()
    base = wid * NCHUNK
    pltpu.sync_copy(src_h.at[pl.ds(base, NCHUNK)], s_v)
    pltpu.sync_copy(dst_h.at[pl.ds(base, NCHUNK)], d_v)
    pltpu.sync_copy(et_h.at[pl.ds(base, NCHUNK)], t_v)

    def zero(i, _):
        ho_v[pl.ds(i * 16, 16)] = jnp.zeros((16,), F32)
        hi_v[pl.ds(i * 16, 16)] = jnp.zeros((16,), F32)
        return 0
    lax.fori_loop(0, H4 // 16, zero, 0)

    ones = jnp.ones((16,), F32)

    def row(ci, _):
        for k8 in range(K // 16):
            sl = pl.ds(k8 * 16, 16)
            s = s_v[ci, sl]
            d = d_v[ci, sl]
            t = t_v[ci, sl]
            g4 = t * NP + s
            d4 = t * NP + d
            gm = jnp.where(t < R, g4, 0)
            plsc.addupdate_scatter(ho_v, [g4], ones)
            plsc.addupdate_scatter(hi_v, [d4], ones)
            s_v[ci, sl] = g4
            d_v[ci, sl] = d4
            t_v[ci, sl] = gm
        return 0
    lax.fori_loop(0, NCHUNK, row, 0)

    pltpu.sync_copy(s_v, g4_out.at[pl.ds(base, NCHUNK)])
    pltpu.sync_copy(d_v, d4_out.at[pl.ds(base, NCHUNK)])
    pltpu.sync_copy(t_v, gm_out.at[pl.ds(base, NCHUNK)])
    pltpu.sync_copy(ho_v, ho_out.at[wid])
    pltpu.sync_copy(hi_v, hi_out.at[wid])


def _k0(src2, dst2, et2):
    out = [jax.ShapeDtypeStruct((NW, H4), F32),
           jax.ShapeDtypeStruct((NW, H4), F32),
           jax.ShapeDtypeStruct((EP // K, K), I32),
           jax.ShapeDtypeStruct((EP // K, K), I32),
           jax.ShapeDtypeStruct((EP // K, K), I32)]
    scratch = [pltpu.VMEM((NCHUNK, K), I32),
               pltpu.VMEM((NCHUNK, K), I32),
               pltpu.VMEM((NCHUNK, K), I32),
               pltpu.VMEM((H4,), F32),
               pltpu.VMEM((H4,), F32)]
    return pl.kernel(_k0_body, out, mesh=_mesh(), scratch_types=scratch,
                     compiler_params=_SC_PARAMS)(
        src2, dst2, et2)


def _t1_body(ho_ref, hi_ref, ao_ref, ai_ref):
    rows = H4 // D
    ri = lax.broadcasted_iota(I32, (rows, D), 0)
    ci = lax.broadcasted_iota(I32, (rows, D), 1)
    mask = ((ri * D + ci) < R * NP).astype(F32)
    dego = jnp.maximum(jnp.sum(ho_ref[...], axis=0), 1.0)
    degi = jnp.maximum(jnp.sum(hi_ref[...], axis=0), 1.0)
    ao_ref[...] = lax.rsqrt(dego) * mask
    ai_ref[...] = lax.rsqrt(degi) * mask


def _t1(ho, hi):
    rows = H4 // D
    return pl.pallas_call(
        _t1_body,
        out_shape=[jax.ShapeDtypeStruct((rows, D), F32),
                   jax.ShapeDtypeStruct((rows, D), F32)],
    )(ho.reshape(NW, rows, D), hi.reshape(NW, rows, D))


def _k2_body(ao_h, ai_h, g4_h, d4_h, coef_out, ao_v, ai_v, g_v, d_v):
    wid = _wid()
    base = wid * NCHUNK
    pltpu.sync_copy(ao_h, ao_v)
    pltpu.sync_copy(ai_h, ai_v)
    pltpu.sync_copy(g4_h.at[pl.ds(base, NCHUNK)], g_v)
    pltpu.sync_copy(d4_h.at[pl.ds(base, NCHUNK)], d_v)

    def row(ci, _):
        for k8 in range(K // 16):
            sl = pl.ds(k8 * 16, 16)
            g4 = g_v[ci, sl]
            d4 = d_v[ci, sl]
            co = plsc.load_gather(ao_v, [g4]) * plsc.load_gather(ai_v, [d4])
            g_v[ci, sl] = plsc.bitcast(co, I32)
        return 0
    lax.fori_loop(0, NCHUNK, row, 0)
    pltpu.sync_copy(g_v, coef_out.at[pl.ds(base, NCHUNK)])


def _k2(ao, ai, g4, d4):
    out = [jax.ShapeDtypeStruct((EP // K, K), I32)]
    scratch = [pltpu.VMEM((H4,), F32),
               pltpu.VMEM((H4,), F32),
               pltpu.VMEM((NCHUNK, K), I32),
               pltpu.VMEM((NCHUNK, K), I32)]
    (coef_bits,) = pl.kernel(_k2_body, out, mesh=_mesh(), scratch_types=scratch,
                             compiler_params=_SC_PARAMS)(ao, ai, g4, d4)
    return coef_bits


def _k3_body(gm_h, dst_h, coef_h, gs_out, ds_out, cs_out,
             s_v, d_v, c_v, og_v, od_v, oc_v, cnt_v):
    wid = _wid()
    base = wid * NCHUNK
    pltpu.sync_copy(gm_h.at[pl.ds(base, NCHUNK)], s_v)
    pltpu.sync_copy(dst_h.at[pl.ds(base, NCHUNK)], d_v)
    pltpu.sync_copy(coef_h.at[pl.ds(base, NCHUNK)], c_v)

    def zero(i, _):
        cnt_v[pl.ds(i * 16, 16)] = jnp.zeros((16,), I32)
        return 0
    lax.fori_loop(0, (R * NP) // 16, zero, 0)

    ones = jnp.ones((16,), I32)

    def hist(ci, _):
        for k8 in range(K // 16):
            g = s_v[ci, pl.ds(k8 * 16, 16)]
            plsc.addupdate_scatter(cnt_v, [g], ones)
        return 0
    lax.fori_loop(0, NCHUNK, hist, 0)

    def scan(i, carry):
        v = cnt_v[pl.ds(i * 16, 16)]
        inc = plsc.cumsum(v)
        cnt_v[pl.ds(i * 16, 16)] = inc - v + carry
        return carry + inc[15]
    lax.fori_loop(0, (R * NP) // 16, scan, jnp.int32(0))

    lane = lax.iota(I32, 16)

    def place(ci, _):
        for k8 in range(K // 16):
            sl = pl.ds(k8 * 16, 16)
            g = s_v[ci, sl]
            d = d_v[ci, sl]
            c = c_v[ci, sl]
            pos = plsc.load_gather(cnt_v, [g])
            rank = jnp.zeros((16,), I32)
            for j in range(16):
                gj = g[j]
                rank = rank + jnp.where(
                    jnp.logical_and(g == gj, lane > j), 1, 0).astype(I32)
            slot = pos + rank
            srow = lax.shift_right_logical(slot, 7)
            scol = lax.bitwise_and(slot, 127)
            plsc.store_scatter(og_v, [srow, scol], g)
            plsc.store_scatter(od_v, [srow, scol], d)
            plsc.store_scatter(oc_v, [srow, scol], c)
            plsc.addupdate_scatter(cnt_v, [g], ones)
        return 0
    lax.fori_loop(0, NCHUNK, place, 0)

    pltpu.sync_copy(og_v, gs_out.at[pl.ds(base, NCHUNK)])
    pltpu.sync_copy(od_v, ds_out.at[pl.ds(base, NCHUNK)])
    pltpu.sync_copy(oc_v, cs_out.at[pl.ds(base, NCHUNK)])


def _k3(gm, dst2, coef):
    out = [jax.ShapeDtypeStruct((EP // K, K), I32),
           jax.ShapeDtypeStruct((EP // K, K), I32),
           jax.ShapeDtypeStruct((EP // K, K), F32)]
    scratch = [pltpu.VMEM((NCHUNK, K), I32),
               pltpu.VMEM((NCHUNK, K), I32),
               pltpu.VMEM((NCHUNK, K), F32),
               pltpu.VMEM((NCHUNK, K), I32),
               pltpu.VMEM((NCHUNK, K), I32),
               pltpu.VMEM((NCHUNK, K), F32),
               pltpu.VMEM((R * NP,), I32)]
    return pl.kernel(_k3_body, out, mesh=_mesh(), scratch_types=scratch,
                     compiler_params=_SC_PARAMS)(gm, dst2, coef)


BM = 2048


def _m_first_body(x_ref, w_ref, out_ref):
    out_ref[0] = jnp.dot(x_ref[...], w_ref[0],
                         preferred_element_type=F32)


def _m_first(x_p, W):
    return pl.pallas_call(
        _m_first_body,
        grid=(R, NP // BM),
        in_specs=[pl.BlockSpec((BM, D), lambda r, i: (i, 0)),
                  pl.BlockSpec((1, D, D), lambda r, i: (r, 0, 0))],
        out_specs=pl.BlockSpec((1, BM, D), lambda r, i: (r, i, 0)),
        out_shape=jax.ShapeDtypeStruct((R, NP, D), F32),
    )(x_p, W)


def _m_mid_body(p0_ref, p1_ref, b_ref, w_ref, out_ref):
    act = jnp.maximum(p0_ref[...] + p1_ref[...] + b_ref[0:1, :], 0.0)
    out_ref[0] = jnp.dot(act, w_ref[0], preferred_element_type=F32)


def _m_mid(p0, p1, bsum, W):
    return pl.pallas_call(
        _m_mid_body,
        grid=(R, NP // BM),
        in_specs=[pl.BlockSpec((BM, D), lambda r, i: (i, 0)),
                  pl.BlockSpec((BM, D), lambda r, i: (i, 0)),
                  pl.BlockSpec((8, D), lambda r, i: (0, 0)),
                  pl.BlockSpec((1, D, D), lambda r, i: (r, 0, 0))],
        out_specs=pl.BlockSpec((1, BM, D), lambda r, i: (r, i, 0)),
        out_shape=jax.ShapeDtypeStruct((R, NP, D), F32),
    )(p0, p1, bsum, W)


SEG = 40
NCH0 = 80
NCH1 = 80


def _s_body(table_h, gm_h, dst_h, coef_h, parts_out,
            acc, g_v, d_v, c_v, r0, r1, gs0, gs1, ss0, ss1):
    rows = [r0, r1]
    gs = [gs0, gs1]
    ss = [ss0, ss1]
    cid = lax.axis_index("c")
    sid = lax.axis_index("s")
    base = jnp.where(cid == 0, sid * NCH0, 16 * NCH0 + sid * NCH1)
    nseg = jnp.where(cid == 0, NCH0 // SEG, NCH1 // SEG)

    def drain_s(b):
        pltpu.make_async_copy(rows[b], acc.at[d_v.at[0]], ss[b]).wait()

    def zero(i, _):
        for k8 in range(D // 16):
            r0[i, pl.ds(k8 * 16, 16)] = jnp.zeros((16,), F32)
        return 0
    lax.fori_loop(0, K, zero, 0)
    for i in range(ROWS_PER_TILE // K):
        pltpu.sync_copy(r0, acc.at[pl.ds(sid * ROWS_PER_TILE + i * K, K)])
    plsc.subcore_barrier()

    def seg_body(seg, _):
        segbase = base + seg * SEG
        pltpu.sync_copy(gm_h.at[pl.ds(segbase, SEG)], g_v)
        pltpu.sync_copy(dst_h.at[pl.ds(segbase, SEG)], d_v)
        pltpu.sync_copy(coef_h.at[pl.ds(segbase, SEG)], c_v)

        @pl.when(seg > 0)
        def _():
            drain_s(0)
            drain_s(1)
        pltpu.async_copy(table_h.at[g_v.at[0]], rows[0], gs[0])

        def pair(sp, _):
            for b in range(2):
                ci = sp * 2 + b
                pltpu.make_async_copy(table_h.at[g_v.at[ci]], rows[b],
                                      gs[b]).wait()

                def mul(q, _):
                    cvec = c_v[ci, pl.ds(q * 16, 16)]
                    for jj in range(16):
                        cs = cvec[jj]
                        for k8 in range(D // 16):
                            sl = pl.ds(k8 * 16, 16)
                            rows[b][q * 16 + jj, sl] = (
                                rows[b][q * 16 + jj, sl] * cs)
                    return 0
                lax.fori_loop(0, K // 16, mul, 0)
                pltpu.async_copy(rows[b], acc.at[d_v.at[ci]], ss[b], add=True)
                if b == 0:
                    @pl.when(sp > 0)
                    def _():
                        drain_s(1)
                    pltpu.async_copy(table_h.at[g_v.at[ci + 1]], rows[1],
                                     gs[1])
                else:
                    @pl.when(sp < SEG // 2 - 1)
                    def _():
                        drain_s(0)
                        pltpu.async_copy(table_h.at[g_v.at[ci + 1]], rows[0],
                                         gs[0])
            return 0
        lax.fori_loop(0, SEG // 2, pair, 0)
        return 0
    lax.fori_loop(0, nseg, seg_body, 0)
    drain_s(0)
    drain_s(1)
    plsc.subcore_barrier()

    sl = pl.ds(sid * ROWS_PER_TILE, ROWS_PER_TILE)
    pltpu.sync_copy(acc.at[sl], parts_out.at[cid].at[sl])


def _s(table, gm, dst2, coef):
    out = [jax.ShapeDtypeStruct((2, NP, D), F32)]
    scratch = ([pltpu.VMEM_SHARED((NP, D), F32),
                pltpu.VMEM((SEG, K), I32),
                pltpu.VMEM((SEG, K), I32),
                pltpu.VMEM((SEG, K), F32)]
               + [pltpu.VMEM((K, D), F32)] * 2
               + [pltpu.SemaphoreType.DMA] * 4)
    (parts,) = pl.kernel(_s_body, out, mesh=_mesh(), scratch_types=scratch,
                         compiler_params=_SC_PARAMS)(table, gm, dst2, coef)
    return parts


def _f_body(p0_ref, p1_ref, b_ref, out_ref):
    out_ref[...] = p0_ref[...] + p1_ref[...] + b_ref[0:1, :]


def _f(p0, p1, bsum):
    return pl.pallas_call(
        _f_body,
        grid=(NP // BM,),
        in_specs=[pl.BlockSpec((BM, D), lambda i: (i, 0)),
                  pl.BlockSpec((BM, D), lambda i: (i, 0)),
                  pl.BlockSpec((8, D), lambda i: (0, 0))],
        out_specs=pl.BlockSpec((BM, D), lambda i: (i, 0)),
        out_shape=jax.ShapeDtypeStruct((NP, D), F32),
    )(p0, p1, bsum)


def kernel(x, edge_index, edge_type, W1, W2, W3, W4, W5, W6,
           b1, b2, b3, b4, b5, b6):
    src = edge_index[0].astype(I32)
    dst = edge_index[1].astype(I32)
    et = edge_type.astype(I32)
    pad = EP - E
    src2 = jnp.concatenate([src, jnp.zeros((pad,), I32)]).reshape(EP // K, K)
    dst2 = jnp.concatenate([dst, jnp.zeros((pad,), I32)]).reshape(EP // K, K)
    et2 = jnp.concatenate([et, jnp.full((pad,), R, I32)]).reshape(EP // K, K)
    x_p = jnp.concatenate([x.astype(F32), jnp.zeros((NP - N, D), F32)])

    ho, hi, g4, d4, gm = _k0(src2, dst2, et2)
    ao2, ai2 = _t1(ho, hi)
    coef_bits = _k2(ao2.reshape(H4), ai2.reshape(H4), g4, d4)
    coef = lax.bitcast_convert_type(coef_bits, F32)
    gm, dst2, coef = _k3(gm, dst2, coef)

    Ws = [W1, W2, W3, W4, W5, W6]
    bs = [b1, b2, b3, b4, b5, b6]
    bsums = [jnp.broadcast_to(b.sum(axis=0)[None, :], (8, D)) for b in bs]

    table = _m_first(x_p, Ws[0]).reshape(R * NP, D)
    parts = _s(table, gm, dst2, coef)
    for l in range(1, 6):
        table = _m_mid(parts[0], parts[1], bsums[l - 1], Ws[l])
        table = table.reshape(R * NP, D)
        parts = _s(table, gm, dst2, coef)
    out = _f(parts[0], parts[1], bsums[5])
    return out[:N]


# speedup vs baseline: 1.1533x; 1.1533x over previous
import functools

import jax
import jax.numpy as jnp
from jax import lax
from jax.experimental import pallas as pl
from jax.experimental.pallas import tpu as pltpu
from jax.experimental.pallas import tpu_sc as plsc

N = 10000
E = 320000
D = 128
R = 3

NP = 10240
NW = 32
NS = 16
EW = 10240
EP = NW * EW
K = 128
NCHUNK = EW // K
ROWS_PER_TILE = NP // NS
H4 = 4 * NP
F32 = jnp.float32
I32 = jnp.int32

_mesh = lambda: plsc.VectorSubcoreMesh(core_axis_name="c", subcore_axis_name="s")
_SC_PARAMS = pltpu.CompilerParams(needs_layout_passes=False)


def _wid():
    return lax.axis_index("s") * 2 + lax.axis_index("c")


def _k0_body(src_h, dst_h, et_h, ho_out, hi_out, g4_out, d4_out, gm_out,
             s_v, d_v, t_v, ho_v, hi_v):
    wid = _wid()
    base = wid * NCHUNK
    pltpu.sync_copy(src_h.at[pl.ds(base, NCHUNK)], s_v)
    pltpu.sync_copy(dst_h.at[pl.ds(base, NCHUNK)], d_v)
    pltpu.sync_copy(et_h.at[pl.ds(base, NCHUNK)], t_v)

    def zero(i, _):
        ho_v[pl.ds(i * 16, 16)] = jnp.zeros((16,), F32)
        hi_v[pl.ds(i * 16, 16)] = jnp.zeros((16,), F32)
        return 0
    lax.fori_loop(0, H4 // 16, zero, 0)

    ones = jnp.ones((16,), F32)

    def row(ci, _):
        for k8 in range(K // 16):
            sl = pl.ds(k8 * 16, 16)
            s = s_v[ci, sl]
            d = d_v[ci, sl]
            t = t_v[ci, sl]
            g4 = t * NP + s
            d4 = t * NP + d
            gm = jnp.where(t < R, g4, 0)
            plsc.addupdate_scatter(ho_v, [g4], ones)
            plsc.addupdate_scatter(hi_v, [d4], ones)
            s_v[ci, sl] = g4
            d_v[ci, sl] = d4
            t_v[ci, sl] = gm
        return 0
    lax.fori_loop(0, NCHUNK, row, 0)

    pltpu.sync_copy(s_v, g4_out.at[pl.ds(base, NCHUNK)])
    pltpu.sync_copy(d_v, d4_out.at[pl.ds(base, NCHUNK)])
    pltpu.sync_copy(t_v, gm_out.at[pl.ds(base, NCHUNK)])
    pltpu.sync_copy(ho_v, ho_out.at[wid])
    pltpu.sync_copy(hi_v, hi_out.at[wid])


def _k0(src2, dst2, et2):
    out = [jax.ShapeDtypeStruct((NW, H4), F32),
           jax.ShapeDtypeStruct((NW, H4), F32),
           jax.ShapeDtypeStruct((EP // K, K), I32),
           jax.ShapeDtypeStruct((EP // K, K), I32),
           jax.ShapeDtypeStruct((EP // K, K), I32)]
    scratch = [pltpu.VMEM((NCHUNK, K), I32),
               pltpu.VMEM((NCHUNK, K), I32),
               pltpu.VMEM((NCHUNK, K), I32),
               pltpu.VMEM((H4,), F32),
               pltpu.VMEM((H4,), F32)]
    return pl.kernel(_k0_body, out, mesh=_mesh(), scratch_types=scratch,
                     compiler_params=_SC_PARAMS)(
        src2, dst2, et2)


def _t1_body(ho_ref, hi_ref, ao_ref, ai_ref):
    rows = H4 // D
    ri = lax.broadcasted_iota(I32, (rows, D), 0)
    ci = lax.broadcasted_iota(I32, (rows, D), 1)
    mask = ((ri * D + ci) < R * NP).astype(F32)
    dego = jnp.maximum(jnp.sum(ho_ref[...], axis=0), 1.0)
    degi = jnp.maximum(jnp.sum(hi_ref[...], axis=0), 1.0)
    ao_ref[...] = lax.rsqrt(dego) * mask
    ai_ref[...] = lax.rsqrt(degi) * mask


def _t1(ho, hi):
    rows = H4 // D
    return pl.pallas_call(
        _t1_body,
        out_shape=[jax.ShapeDtypeStruct((rows, D), F32),
                   jax.ShapeDtypeStruct((rows, D), F32)],
    )(ho.reshape(NW, rows, D), hi.reshape(NW, rows, D))


def _k2_body(ao_h, ai_h, g4_h, d4_h, coef_out, ao_v, ai_v, g_v, d_v):
    wid = _wid()
    base = wid * NCHUNK
    pltpu.sync_copy(ao_h, ao_v)
    pltpu.sync_copy(ai_h, ai_v)
    pltpu.sync_copy(g4_h.at[pl.ds(base, NCHUNK)], g_v)
    pltpu.sync_copy(d4_h.at[pl.ds(base, NCHUNK)], d_v)

    def row(ci, _):
        for k8 in range(K // 16):
            sl = pl.ds(k8 * 16, 16)
            g4 = g_v[ci, sl]
            d4 = d_v[ci, sl]
            co = plsc.load_gather(ao_v, [g4]) * plsc.load_gather(ai_v, [d4])
            g_v[ci, sl] = plsc.bitcast(co, I32)
        return 0
    lax.fori_loop(0, NCHUNK, row, 0)
    pltpu.sync_copy(g_v, coef_out.at[pl.ds(base, NCHUNK)])


def _k2(ao, ai, g4, d4):
    out = [jax.ShapeDtypeStruct((EP // K, K), I32)]
    scratch = [pltpu.VMEM((H4,), F32),
               pltpu.VMEM((H4,), F32),
               pltpu.VMEM((NCHUNK, K), I32),
               pltpu.VMEM((NCHUNK, K), I32)]
    (coef_bits,) = pl.kernel(_k2_body, out, mesh=_mesh(), scratch_types=scratch,
                             compiler_params=_SC_PARAMS)(ao, ai, g4, d4)
    return coef_bits


def _k3_body(gm_h, dst_h, coef_h, gs_out, ds_out, cs_out,
             s_v, d_v, c_v, og_v, od_v, oc_v, cnt_v):
    wid = _wid()
    base = wid * NCHUNK
    pltpu.sync_copy(gm_h.at[pl.ds(base, NCHUNK)], s_v)
    pltpu.sync_copy(dst_h.at[pl.ds(base, NCHUNK)], d_v)
    pltpu.sync_copy(coef_h.at[pl.ds(base, NCHUNK)], c_v)

    def zero(i, _):
        cnt_v[pl.ds(i * 16, 16)] = jnp.zeros((16,), I32)
        return 0
    lax.fori_loop(0, (R * NP) // 16, zero, 0)

    ones = jnp.ones((16,), I32)

    def hist(ci, _):
        for k8 in range(K // 16):
            g = s_v[ci, pl.ds(k8 * 16, 16)]
            plsc.addupdate_scatter(cnt_v, [g], ones)
        return 0
    lax.fori_loop(0, NCHUNK, hist, 0)

    def scan(i, carry):
        v = cnt_v[pl.ds(i * 16, 16)]
        inc = plsc.cumsum(v)
        cnt_v[pl.ds(i * 16, 16)] = inc - v + carry
        return carry + inc[15]
    lax.fori_loop(0, (R * NP) // 16, scan, jnp.int32(0))

    lane = lax.iota(I32, 16)

    def place(ci, _):
        for k8 in range(K // 16):
            sl = pl.ds(k8 * 16, 16)
            g = s_v[ci, sl]
            d = d_v[ci, sl]
            c = c_v[ci, sl]
            pos = plsc.load_gather(cnt_v, [g])
            rank = jnp.zeros((16,), I32)
            for j in range(16):
                gj = g[j]
                rank = rank + jnp.where(
                    jnp.logical_and(g == gj, lane > j), 1, 0).astype(I32)
            slot = pos + rank
            srow = lax.shift_right_logical(slot, 7)
            scol = lax.bitwise_and(slot, 127)
            plsc.store_scatter(og_v, [srow, scol], g)
            plsc.store_scatter(od_v, [srow, scol], d)
            plsc.store_scatter(oc_v, [srow, scol], c)
            plsc.addupdate_scatter(cnt_v, [g], ones)
        return 0
    lax.fori_loop(0, NCHUNK, place, 0)

    pltpu.sync_copy(og_v, gs_out.at[pl.ds(base, NCHUNK)])
    pltpu.sync_copy(od_v, ds_out.at[pl.ds(base, NCHUNK)])
    pltpu.sync_copy(oc_v, cs_out.at[pl.ds(base, NCHUNK)])


def _k3(gm, dst2, coef):
    out = [jax.ShapeDtypeStruct((EP // K, K), I32),
           jax.ShapeDtypeStruct((EP // K, K), I32),
           jax.ShapeDtypeStruct((EP // K, K), F32)]
    scratch = [pltpu.VMEM((NCHUNK, K), I32),
               pltpu.VMEM((NCHUNK, K), I32),
               pltpu.VMEM((NCHUNK, K), F32),
               pltpu.VMEM((NCHUNK, K), I32),
               pltpu.VMEM((NCHUNK, K), I32),
               pltpu.VMEM((NCHUNK, K), F32),
               pltpu.VMEM((R * NP,), I32)]
    return pl.kernel(_k3_body, out, mesh=_mesh(), scratch_types=scratch,
                     compiler_params=_SC_PARAMS)(gm, dst2, coef)


BM = 2048


def _m_first_body(x_ref, w_ref, out_ref):
    out_ref[0] = jnp.dot(x_ref[...], w_ref[0],
                         preferred_element_type=F32)


def _m_first(x_p, W):
    return pl.pallas_call(
        _m_first_body,
        grid=(R, NP // BM),
        in_specs=[pl.BlockSpec((BM, D), lambda r, i: (i, 0)),
                  pl.BlockSpec((1, D, D), lambda r, i: (r, 0, 0))],
        out_specs=pl.BlockSpec((1, BM, D), lambda r, i: (r, i, 0)),
        out_shape=jax.ShapeDtypeStruct((R, NP, D), F32),
    )(x_p, W)


def _m_mid_body(p0_ref, p1_ref, b_ref, w_ref, out_ref):
    act = jnp.maximum(p0_ref[...] + p1_ref[...] + b_ref[0:1, :], 0.0)
    out_ref[0] = jnp.dot(act, w_ref[0], preferred_element_type=F32)


def _m_mid(p0, p1, bsum, W):
    return pl.pallas_call(
        _m_mid_body,
        grid=(R, NP // BM),
        in_specs=[pl.BlockSpec((BM, D), lambda r, i: (i, 0)),
                  pl.BlockSpec((BM, D), lambda r, i: (i, 0)),
                  pl.BlockSpec((8, D), lambda r, i: (0, 0)),
                  pl.BlockSpec((1, D, D), lambda r, i: (r, 0, 0))],
        out_specs=pl.BlockSpec((1, BM, D), lambda r, i: (r, i, 0)),
        out_shape=jax.ShapeDtypeStruct((R, NP, D), F32),
    )(p0, p1, bsum, W)


SEG = 40
NCH0 = 120
NCH1 = 40


def _s_body(table_h, gm_h, dst_h, coef_h, parts_out,
            acc, g_v, d_v, c_v, r0, r1, gs0, gs1, ss0, ss1):
    rows = [r0, r1]
    gs = [gs0, gs1]
    ss = [ss0, ss1]
    cid = lax.axis_index("c")
    sid = lax.axis_index("s")
    base = jnp.where(cid == 0, sid * NCH0, 16 * NCH0 + sid * NCH1)
    nseg = jnp.where(cid == 0, NCH0 // SEG, NCH1 // SEG)

    def drain_s(b):
        pltpu.make_async_copy(rows[b], acc.at[d_v.at[0]], ss[b]).wait()

    def zero(i, _):
        for k8 in range(D // 16):
            r0[i, pl.ds(k8 * 16, 16)] = jnp.zeros((16,), F32)
        return 0
    lax.fori_loop(0, K, zero, 0)
    for i in range(ROWS_PER_TILE // K):
        pltpu.sync_copy(r0, acc.at[pl.ds(sid * ROWS_PER_TILE + i * K, K)])
    plsc.subcore_barrier()

    def seg_body(seg, _):
        segbase = base + seg * SEG
        pltpu.sync_copy(gm_h.at[pl.ds(segbase, SEG)], g_v)
        pltpu.sync_copy(dst_h.at[pl.ds(segbase, SEG)], d_v)
        pltpu.sync_copy(coef_h.at[pl.ds(segbase, SEG)], c_v)

        @pl.when(seg > 0)
        def _():
            drain_s(0)
            drain_s(1)
        pltpu.async_copy(table_h.at[g_v.at[0]], rows[0], gs[0])

        def pair(sp, _):
            for b in range(2):
                ci = sp * 2 + b
                pltpu.make_async_copy(table_h.at[g_v.at[ci]], rows[b],
                                      gs[b]).wait()

                def mul(q, _):
                    cvec = c_v[ci, pl.ds(q * 16, 16)]
                    for jj in range(16):
                        cs = cvec[jj]
                        for k8 in range(D // 16):
                            sl = pl.ds(k8 * 16, 16)
                            rows[b][q * 16 + jj, sl] = (
                                rows[b][q * 16 + jj, sl] * cs)
                    return 0
                lax.fori_loop(0, K // 16, mul, 0)
                pltpu.async_copy(rows[b], acc.at[d_v.at[ci]], ss[b], add=True)
                if b == 0:
                    @pl.when(sp > 0)
                    def _():
                        drain_s(1)
                    pltpu.async_copy(table_h.at[g_v.at[ci + 1]], rows[1],
                                     gs[1])
                else:
                    @pl.when(sp < SEG // 2 - 1)
                    def _():
                        drain_s(0)
                        pltpu.async_copy(table_h.at[g_v.at[ci + 1]], rows[0],
                                         gs[0])
            return 0
        lax.fori_loop(0, SEG // 2, pair, 0)
        return 0
    lax.fori_loop(0, nseg, seg_body, 0)
    drain_s(0)
    drain_s(1)
    plsc.subcore_barrier()

    sl = pl.ds(sid * ROWS_PER_TILE, ROWS_PER_TILE)
    pltpu.sync_copy(acc.at[sl], parts_out.at[cid].at[sl])


def _s(table, gm, dst2, coef):
    out = [jax.ShapeDtypeStruct((2, NP, D), F32)]
    scratch = ([pltpu.VMEM_SHARED((NP, D), F32),
                pltpu.VMEM((SEG, K), I32),
                pltpu.VMEM((SEG, K), I32),
                pltpu.VMEM((SEG, K), F32)]
               + [pltpu.VMEM((K, D), F32)] * 2
               + [pltpu.SemaphoreType.DMA] * 4)
    (parts,) = pl.kernel(_s_body, out, mesh=_mesh(), scratch_types=scratch,
                         compiler_params=_SC_PARAMS)(table, gm, dst2, coef)
    return parts


def _f_body(p0_ref, p1_ref, b_ref, out_ref):
    out_ref[...] = p0_ref[...] + p1_ref[...] + b_ref[0:1, :]


def _f(p0, p1, bsum):
    return pl.pallas_call(
        _f_body,
        grid=(NP // BM,),
        in_specs=[pl.BlockSpec((BM, D), lambda i: (i, 0)),
                  pl.BlockSpec((BM, D), lambda i: (i, 0)),
                  pl.BlockSpec((8, D), lambda i: (0, 0))],
        out_specs=pl.BlockSpec((BM, D), lambda i: (i, 0)),
        out_shape=jax.ShapeDtypeStruct((NP, D), F32),
    )(p0, p1, bsum)


def kernel(x, edge_index, edge_type, W1, W2, W3, W4, W5, W6,
           b1, b2, b3, b4, b5, b6):
    src = edge_index[0].astype(I32)
    dst = edge_index[1].astype(I32)
    et = edge_type.astype(I32)
    pad = EP - E
    src2 = jnp.concatenate([src, jnp.zeros((pad,), I32)]).reshape(EP // K, K)
    dst2 = jnp.concatenate([dst, jnp.zeros((pad,), I32)]).reshape(EP // K, K)
    et2 = jnp.concatenate([et, jnp.full((pad,), R, I32)]).reshape(EP // K, K)
    x_p = jnp.concatenate([x.astype(F32), jnp.zeros((NP - N, D), F32)])

    ho, hi, g4, d4, gm = _k0(src2, dst2, et2)
    ao2, ai2 = _t1(ho, hi)
    coef_bits = _k2(ao2.reshape(H4), ai2.reshape(H4), g4, d4)
    coef = lax.bitcast_convert_type(coef_bits, F32)
    gm, dst2, coef = _k3(gm, dst2, coef)

    Ws = [W1, W2, W3, W4, W5, W6]
    bs = [b1, b2, b3, b4, b5, b6]
    bsums = [jnp.broadcast_to(b.sum(axis=0)[None, :], (8, D)) for b in bs]

    table = _m_first(x_p, Ws[0]).reshape(R * NP, D)
    parts = _s(table, gm, dst2, coef)
    for l in range(1, 6):
        table = _m_mid(parts[0], parts[1], bsums[l - 1], Ws[l])
        table = table.reshape(R * NP, D)
        parts = _s(table, gm, dst2, coef)
    out = _f(parts[0], parts[1], bsums[5])
    return out[:N]


# speedup vs baseline: 1.1564x; 1.0027x over previous
import functools

import jax
import jax.numpy as jnp
from jax import lax
from jax.experimental import pallas as pl
from jax.experimental.pallas import tpu as pltpu
from jax.experimental.pallas import tpu_sc as plsc

N = 10000
E = 320000
D = 128
R = 3

NP = 10240
NW = 32
NS = 16
EW = 10240
EP = NW * EW
K = 128
NCHUNK = EW // K
ROWS_PER_TILE = NP // NS
H4 = 4 * NP
F32 = jnp.float32
I32 = jnp.int32

_mesh = lambda: plsc.VectorSubcoreMesh(core_axis_name="c", subcore_axis_name="s")
_SC_PARAMS = pltpu.CompilerParams(needs_layout_passes=False)


def _wid():
    return lax.axis_index("s") * 2 + lax.axis_index("c")


def _k0_body(src_h, dst_h, et_h, ho_out, hi_out, g4_out, d4_out, gm_out,
             s_v, d_v, t_v, ho_v, hi_v):
    wid = _wid()
    base = wid * NCHUNK
    pltpu.sync_copy(src_h.at[pl.ds(base, NCHUNK)], s_v)
    pltpu.sync_copy(dst_h.at[pl.ds(base, NCHUNK)], d_v)
    pltpu.sync_copy(et_h.at[pl.ds(base, NCHUNK)], t_v)

    def zero(i, _):
        ho_v[pl.ds(i * 16, 16)] = jnp.zeros((16,), F32)
        hi_v[pl.ds(i * 16, 16)] = jnp.zeros((16,), F32)
        return 0
    lax.fori_loop(0, H4 // 16, zero, 0)

    ones = jnp.ones((16,), F32)

    def row(ci, _):
        for k8 in range(K // 16):
            sl = pl.ds(k8 * 16, 16)
            s = s_v[ci, sl]
            d = d_v[ci, sl]
            t = t_v[ci, sl]
            g4 = t * NP + s
            d4 = t * NP + d
            gm = jnp.where(t < R, g4, 0)
            plsc.addupdate_scatter(ho_v, [g4], ones)
            plsc.addupdate_scatter(hi_v, [d4], ones)
            s_v[ci, sl] = g4
            d_v[ci, sl] = d4
            t_v[ci, sl] = gm
        return 0
    lax.fori_loop(0, NCHUNK, row, 0)

    pltpu.sync_copy(s_v, g4_out.at[pl.ds(base, NCHUNK)])
    pltpu.sync_copy(d_v, d4_out.at[pl.ds(base, NCHUNK)])
    pltpu.sync_copy(t_v, gm_out.at[pl.ds(base, NCHUNK)])
    pltpu.sync_copy(ho_v, ho_out.at[wid])
    pltpu.sync_copy(hi_v, hi_out.at[wid])


def _k0(src2, dst2, et2):
    out = [jax.ShapeDtypeStruct((NW, H4), F32),
           jax.ShapeDtypeStruct((NW, H4), F32),
           jax.ShapeDtypeStruct((EP // K, K), I32),
           jax.ShapeDtypeStruct((EP // K, K), I32),
           jax.ShapeDtypeStruct((EP // K, K), I32)]
    scratch = [pltpu.VMEM((NCHUNK, K), I32),
               pltpu.VMEM((NCHUNK, K), I32),
               pltpu.VMEM((NCHUNK, K), I32),
               pltpu.VMEM((H4,), F32),
               pltpu.VMEM((H4,), F32)]
    return pl.kernel(_k0_body, out, mesh=_mesh(), scratch_types=scratch,
                     compiler_params=_SC_PARAMS)(
        src2, dst2, et2)


def _t1_body(ho_ref, hi_ref, ao_ref, ai_ref):
    rows = H4 // D
    ri = lax.broadcasted_iota(I32, (rows, D), 0)
    ci = lax.broadcasted_iota(I32, (rows, D), 1)
    mask = ((ri * D + ci) < R * NP).astype(F32)
    dego = jnp.maximum(jnp.sum(ho_ref[...], axis=0), 1.0)
    degi = jnp.maximum(jnp.sum(hi_ref[...], axis=0), 1.0)
    ao_ref[...] = lax.rsqrt(dego) * mask
    ai_ref[...] = lax.rsqrt(degi) * mask


def _t1(ho, hi):
    rows = H4 // D
    return pl.pallas_call(
        _t1_body,
        out_shape=[jax.ShapeDtypeStruct((rows, D), F32),
                   jax.ShapeDtypeStruct((rows, D), F32)],
    )(ho.reshape(NW, rows, D), hi.reshape(NW, rows, D))


def _k2_body(ao_h, ai_h, g4_h, d4_h, coef_out, ao_v, ai_v, g_v, d_v):
    wid = _wid()
    base = wid * NCHUNK
    pltpu.sync_copy(ao_h, ao_v)
    pltpu.sync_copy(ai_h, ai_v)
    pltpu.sync_copy(g4_h.at[pl.ds(base, NCHUNK)], g_v)
    pltpu.sync_copy(d4_h.at[pl.ds(base, NCHUNK)], d_v)

    def row(ci, _):
        for k8 in range(K // 16):
            sl = pl.ds(k8 * 16, 16)
            g4 = g_v[ci, sl]
            d4 = d_v[ci, sl]
            co = plsc.load_gather(ao_v, [g4]) * plsc.load_gather(ai_v, [d4])
            g_v[ci, sl] = plsc.bitcast(co, I32)
        return 0
    lax.fori_loop(0, NCHUNK, row, 0)
    pltpu.sync_copy(g_v, coef_out.at[pl.ds(base, NCHUNK)])


def _k2(ao, ai, g4, d4):
    out = [jax.ShapeDtypeStruct((EP // K, K), I32)]
    scratch = [pltpu.VMEM((H4,), F32),
               pltpu.VMEM((H4,), F32),
               pltpu.VMEM((NCHUNK, K), I32),
               pltpu.VMEM((NCHUNK, K), I32)]
    (coef_bits,) = pl.kernel(_k2_body, out, mesh=_mesh(), scratch_types=scratch,
                             compiler_params=_SC_PARAMS)(ao, ai, g4, d4)
    return coef_bits


def _k3_body(gm_h, dst_h, coef_h, gs_out, ds_out, cs_out,
             s_v, d_v, c_v, og_v, od_v, oc_v, cnt_v):
    wid = _wid()
    base = wid * NCHUNK
    pltpu.sync_copy(gm_h.at[pl.ds(base, NCHUNK)], s_v)
    pltpu.sync_copy(dst_h.at[pl.ds(base, NCHUNK)], d_v)
    pltpu.sync_copy(coef_h.at[pl.ds(base, NCHUNK)], c_v)

    def zero(i, _):
        cnt_v[pl.ds(i * 16, 16)] = jnp.zeros((16,), I32)
        return 0
    lax.fori_loop(0, (R * NP) // 16, zero, 0)

    ones = jnp.ones((16,), I32)

    def hist(ci, _):
        for k8 in range(K // 16):
            g = s_v[ci, pl.ds(k8 * 16, 16)]
            plsc.addupdate_scatter(cnt_v, [g], ones)
        return 0
    lax.fori_loop(0, NCHUNK, hist, 0)

    def scan(i, carry):
        v = cnt_v[pl.ds(i * 16, 16)]
        inc = plsc.cumsum(v)
        cnt_v[pl.ds(i * 16, 16)] = inc - v + carry
        return carry + inc[15]
    lax.fori_loop(0, (R * NP) // 16, scan, jnp.int32(0))

    lane = lax.iota(I32, 16)

    def place(ci, _):
        for k8 in range(K // 16):
            sl = pl.ds(k8 * 16, 16)
            g = s_v[ci, sl]
            d = d_v[ci, sl]
            c = c_v[ci, sl]
            pos = plsc.load_gather(cnt_v, [g])
            rank = jnp.zeros((16,), I32)
            for j in range(16):
                gj = g[j]
                rank = rank + jnp.where(
                    jnp.logical_and(g == gj, lane > j), 1, 0).astype(I32)
            slot = pos + rank
            srow = lax.shift_right_logical(slot, 7)
            scol = lax.bitwise_and(slot, 127)
            plsc.store_scatter(og_v, [srow, scol], g)
            plsc.store_scatter(od_v, [srow, scol], d)
            plsc.store_scatter(oc_v, [srow, scol], c)
            plsc.addupdate_scatter(cnt_v, [g], ones)
        return 0
    lax.fori_loop(0, NCHUNK, place, 0)

    pltpu.sync_copy(og_v, gs_out.at[pl.ds(base, NCHUNK)])
    pltpu.sync_copy(od_v, ds_out.at[pl.ds(base, NCHUNK)])
    pltpu.sync_copy(oc_v, cs_out.at[pl.ds(base, NCHUNK)])


def _k3(gm, dst2, coef):
    out = [jax.ShapeDtypeStruct((EP // K, K), I32),
           jax.ShapeDtypeStruct((EP // K, K), I32),
           jax.ShapeDtypeStruct((EP // K, K), F32)]
    scratch = [pltpu.VMEM((NCHUNK, K), I32),
               pltpu.VMEM((NCHUNK, K), I32),
               pltpu.VMEM((NCHUNK, K), F32),
               pltpu.VMEM((NCHUNK, K), I32),
               pltpu.VMEM((NCHUNK, K), I32),
               pltpu.VMEM((NCHUNK, K), F32),
               pltpu.VMEM((R * NP,), I32)]
    return pl.kernel(_k3_body, out, mesh=_mesh(), scratch_types=scratch,
                     compiler_params=_SC_PARAMS)(gm, dst2, coef)


BM = 2048


def _m_first_body(x_ref, w_ref, out_ref):
    out_ref[0] = jnp.dot(x_ref[...], w_ref[0],
                         preferred_element_type=F32)


def _m_first(x_p, W):
    return pl.pallas_call(
        _m_first_body,
        grid=(R, NP // BM),
        in_specs=[pl.BlockSpec((BM, D), lambda r, i: (i, 0)),
                  pl.BlockSpec((1, D, D), lambda r, i: (r, 0, 0))],
        out_specs=pl.BlockSpec((1, BM, D), lambda r, i: (r, i, 0)),
        out_shape=jax.ShapeDtypeStruct((R, NP, D), F32),
    )(x_p, W)


def _m_mid_body(p0_ref, p1_ref, b_ref, w_ref, out_ref):
    act = jnp.maximum(p0_ref[...] + p1_ref[...] + b_ref[0:1, :], 0.0)
    out_ref[0] = jnp.dot(act, w_ref[0], preferred_element_type=F32)


def _m_mid(p0, p1, bsum, W):
    return pl.pallas_call(
        _m_mid_body,
        grid=(R, NP // BM),
        in_specs=[pl.BlockSpec((BM, D), lambda r, i: (i, 0)),
                  pl.BlockSpec((BM, D), lambda r, i: (i, 0)),
                  pl.BlockSpec((8, D), lambda r, i: (0, 0)),
                  pl.BlockSpec((1, D, D), lambda r, i: (r, 0, 0))],
        out_specs=pl.BlockSpec((1, BM, D), lambda r, i: (r, i, 0)),
        out_shape=jax.ShapeDtypeStruct((R, NP, D), F32),
    )(p0, p1, bsum, W)


SEG = 16
NCH0 = 128
NCH1 = 32


def _s_body(table_h, gm_h, dst_h, coef_h, parts_out,
            acc, g_v, d_v, c_v, r0, r1, gs0, gs1, ss0, ss1):
    rows = [r0, r1]
    gs = [gs0, gs1]
    ss = [ss0, ss1]
    cid = lax.axis_index("c")
    sid = lax.axis_index("s")
    base = jnp.where(cid == 0, sid * NCH0, 16 * NCH0 + sid * NCH1)
    nseg = jnp.where(cid == 0, NCH0 // SEG, NCH1 // SEG)

    def drain_s(b):
        pltpu.make_async_copy(rows[b], acc.at[d_v.at[0]], ss[b]).wait()

    def zero(i, _):
        for k8 in range(D // 16):
            r0[i, pl.ds(k8 * 16, 16)] = jnp.zeros((16,), F32)
        return 0
    lax.fori_loop(0, K, zero, 0)
    for i in range(ROWS_PER_TILE // K):
        pltpu.sync_copy(r0, acc.at[pl.ds(sid * ROWS_PER_TILE + i * K, K)])
    plsc.subcore_barrier()

    def seg_body(seg, _):
        segbase = base + seg * SEG
        pltpu.sync_copy(gm_h.at[pl.ds(segbase, SEG)], g_v)
        pltpu.sync_copy(dst_h.at[pl.ds(segbase, SEG)], d_v)
        pltpu.sync_copy(coef_h.at[pl.ds(segbase, SEG)], c_v)

        @pl.when(seg > 0)
        def _():
            drain_s(0)
            drain_s(1)
        pltpu.async_copy(table_h.at[g_v.at[0]], rows[0], gs[0])

        def pair(sp, _):
            for b in range(2):
                ci = sp * 2 + b
                pltpu.make_async_copy(table_h.at[g_v.at[ci]], rows[b],
                                      gs[b]).wait()

                def mul(q, _):
                    cvec = c_v[ci, pl.ds(q * 16, 16)]
                    for jj in range(16):
                        cs = cvec[jj]
                        for k8 in range(D // 16):
                            sl = pl.ds(k8 * 16, 16)
                            rows[b][q * 16 + jj, sl] = (
                                rows[b][q * 16 + jj, sl] * cs)
                    return 0
                lax.fori_loop(0, K // 16, mul, 0)
                pltpu.async_copy(rows[b], acc.at[d_v.at[ci]], ss[b], add=True)
                if b == 0:
                    @pl.when(sp > 0)
                    def _():
                        drain_s(1)
                    pltpu.async_copy(table_h.at[g_v.at[ci + 1]], rows[1],
                                     gs[1])
                else:
                    @pl.when(sp < SEG // 2 - 1)
                    def _():
                        drain_s(0)
                        pltpu.async_copy(table_h.at[g_v.at[ci + 1]], rows[0],
                                         gs[0])
            return 0
        lax.fori_loop(0, SEG // 2, pair, 0)
        return 0
    lax.fori_loop(0, nseg, seg_body, 0)
    drain_s(0)
    drain_s(1)
    plsc.subcore_barrier()

    sl = pl.ds(sid * ROWS_PER_TILE, ROWS_PER_TILE)
    pltpu.sync_copy(acc.at[sl], parts_out.at[cid].at[sl])


def _s(table, gm, dst2, coef):
    out = [jax.ShapeDtypeStruct((2, NP, D), F32)]
    scratch = ([pltpu.VMEM_SHARED((NP, D), F32),
                pltpu.VMEM((SEG, K), I32),
                pltpu.VMEM((SEG, K), I32),
                pltpu.VMEM((SEG, K), F32)]
               + [pltpu.VMEM((K, D), F32)] * 2
               + [pltpu.SemaphoreType.DMA] * 4)
    (parts,) = pl.kernel(_s_body, out, mesh=_mesh(), scratch_types=scratch,
                         compiler_params=_SC_PARAMS)(table, gm, dst2, coef)
    return parts


def _f_body(p0_ref, p1_ref, b_ref, out_ref):
    out_ref[...] = p0_ref[...] + p1_ref[...] + b_ref[0:1, :]


def _f(p0, p1, bsum):
    return pl.pallas_call(
        _f_body,
        grid=(NP // BM,),
        in_specs=[pl.BlockSpec((BM, D), lambda i: (i, 0)),
                  pl.BlockSpec((BM, D), lambda i: (i, 0)),
                  pl.BlockSpec((8, D), lambda i: (0, 0))],
        out_specs=pl.BlockSpec((BM, D), lambda i: (i, 0)),
        out_shape=jax.ShapeDtypeStruct((NP, D), F32),
    )(p0, p1, bsum)


def kernel(x, edge_index, edge_type, W1, W2, W3, W4, W5, W6,
           b1, b2, b3, b4, b5, b6):
    src = edge_index[0].astype(I32)
    dst = edge_index[1].astype(I32)
    et = edge_type.astype(I32)
    pad = EP - E
    src2 = jnp.concatenate([src, jnp.zeros((pad,), I32)]).reshape(EP // K, K)
    dst2 = jnp.concatenate([dst, jnp.zeros((pad,), I32)]).reshape(EP // K, K)
    et2 = jnp.concatenate([et, jnp.full((pad,), R, I32)]).reshape(EP // K, K)
    x_p = jnp.concatenate([x.astype(F32), jnp.zeros((NP - N, D), F32)])

    ho, hi, g4, d4, gm = _k0(src2, dst2, et2)
    ao2, ai2 = _t1(ho, hi)
    coef_bits = _k2(ao2.reshape(H4), ai2.reshape(H4), g4, d4)
    coef = lax.bitcast_convert_type(coef_bits, F32)
    gm, dst2, coef = _k3(gm, dst2, coef)

    Ws = [W1, W2, W3, W4, W5, W6]
    bs = [b1, b2, b3, b4, b5, b6]
    bsums = [jnp.broadcast_to(b.sum(axis=0)[None, :], (8, D)) for b in bs]

    table = _m_first(x_p, Ws[0]).reshape(R * NP, D)
    parts = _s(table, gm, dst2, coef)
    for l in range(1, 6):
        table = _m_mid(parts[0], parts[1], bsums[l - 1], Ws[l])
        table = table.reshape(R * NP, D)
        parts = _s(table, gm, dst2, coef)
    out = _f(parts[0], parts[1], bsums[5])
    return out[:N]
